# baseline (device time: 16171 ns/iter reference)
import functools

import jax
import jax.numpy as jnp
from jax import lax
from jax.experimental import pallas as pl
from jax.experimental.pallas import tpu as pltpu

N_DEV = 4
C = 4


def kernel(x):
    m, n_per = x.shape
    mc = m // C

    def body(x_ref, out_ref, stage_ref, e_ref, comm_ref,
             in_sems, out_sems, send_sems, recv_sems):
        my_pos = lax.axis_index("i")
        peers = [lax.rem(my_pos + d, N_DEV) for d in range(1, N_DEV)]

        barrier_sem = pltpu.get_barrier_semaphore()
        for peer in peers:
            pl.semaphore_signal(
                barrier_sem, inc=1,
                device_id=(peer,), device_id_type=pl.DeviceIdType.MESH,
            )

        def copy_in(c):
            return pltpu.make_async_copy(
                x_ref.at[pl.ds(c * mc, mc), :],
                stage_ref.at[c % 2],
                in_sems.at[c % 2],
            )

        copy_in(0).start()
        rdmas = [[] for _ in range(C)]
        for c in range(C):
            if c + 1 < C:
                copy_in(c + 1).start()
            copy_in(c).wait()
            ev = jnp.exp(stage_ref[c % 2, :, :])
            e_ref[pl.ds(c * mc, mc), :] = ev
            comm_ref[c, N_DEV - 1, :] = jnp.sum(ev, axis=1)
            if c == 0:
                pl.semaphore_wait(barrier_sem, N_DEV - 1)
            for d in range(1, N_DEV):
                rdma = pltpu.make_async_remote_copy(
                    src_ref=comm_ref.at[c, N_DEV - 1],
                    dst_ref=comm_ref.at[c, d - 1],
                    send_sem=send_sems.at[c, d - 1],
                    recv_sem=recv_sems.at[c, d - 1],
                    device_id=(peers[d - 1],),
                    device_id_type=pl.DeviceIdType.MESH,
                )
                rdma.start()
                rdmas[c].append(rdma)

        out_copies = []
        for c in range(C):
            for r in rdmas[c]:
                r.wait_recv()
            g_sum = jnp.sum(comm_ref[c, :, :], axis=0)
            rows = pl.ds(c * mc, mc)
            e_ref[rows, :] = e_ref[rows, :] * (1.0 / g_sum)[:, None]
            cp = pltpu.make_async_copy(
                e_ref.at[rows, :], out_ref.at[rows, :], out_sems.at[c]
            )
            cp.start()
            out_copies.append(cp)

        for c in range(C):
            for r in rdmas[c]:
                r.wait_send()

        @functools.partial(
            pl.run_scoped, second_barrier=pltpu.SemaphoreType.REGULAR
        )
        def _(second_barrier):
            for peer in peers:
                pl.semaphore_signal(
                    second_barrier, inc=1,
                    device_id=(peer,), device_id_type=pl.DeviceIdType.MESH,
                )
            pl.semaphore_wait(second_barrier, N_DEV - 1)

        for cp in out_copies:
            cp.wait()

    return pl.pallas_call(
        body,
        out_shape=jax.ShapeDtypeStruct((m, n_per), jnp.float32),
        in_specs=[pl.BlockSpec(memory_space=pl.ANY)],
        out_specs=pl.BlockSpec(memory_space=pl.ANY),
        scratch_shapes=[
            pltpu.VMEM((2, mc, n_per), jnp.float32),
            pltpu.VMEM((m, n_per), jnp.float32),
            pltpu.VMEM((C, N_DEV, mc), jnp.float32),
            pltpu.SemaphoreType.DMA((2,)),
            pltpu.SemaphoreType.DMA((C,)),
            pltpu.SemaphoreType.DMA((C, N_DEV - 1)),
            pltpu.SemaphoreType.DMA((C, N_DEV - 1)),
        ],
        compiler_params=pltpu.CompilerParams(collective_id=0),
    )(x)


# device time: 13729 ns/iter; 1.1779x vs baseline; 1.1779x over previous
import functools

import jax
import jax.numpy as jnp
from jax import lax
from jax.experimental import pallas as pl
from jax.experimental.pallas import tpu as pltpu

N_DEV = 4
C = 4


def kernel(x):
    m, n_per = x.shape
    mc = m // C

    def body(x_ref, out_ref, stage_ref, comm_ref,
             in_sems, send_sems, recv_sems):
        my_pos = lax.axis_index("i")
        peers = [lax.rem(my_pos + d, N_DEV) for d in range(1, N_DEV)]

        barrier_sem = pltpu.get_barrier_semaphore()
        for peer in peers:
            pl.semaphore_signal(
                barrier_sem, inc=1,
                device_id=(peer,), device_id_type=pl.DeviceIdType.MESH,
            )

        def copy_in(c):
            return pltpu.make_async_copy(
                x_ref.at[pl.ds(c * mc, mc), :],
                stage_ref.at[c % 2],
                in_sems.at[c % 2],
            )

        copy_in(0).start()
        rdmas = [[] for _ in range(C)]
        for c in range(C):
            if c + 1 < C:
                copy_in(c + 1).start()
            copy_in(c).wait()
            ev = jnp.exp(stage_ref[c % 2, :, :])
            out_ref[pl.ds(c * mc, mc), :] = ev
            comm_ref[c, N_DEV - 1, :] = jnp.sum(ev, axis=1)
            if c == 0:
                pl.semaphore_wait(barrier_sem, N_DEV - 1)
            for d in range(1, N_DEV):
                rdma = pltpu.make_async_remote_copy(
                    src_ref=comm_ref.at[c, N_DEV - 1],
                    dst_ref=comm_ref.at[c, d - 1],
                    send_sem=send_sems.at[c, d - 1],
                    recv_sem=recv_sems.at[c, d - 1],
                    device_id=(peers[d - 1],),
                    device_id_type=pl.DeviceIdType.MESH,
                )
                rdma.start()
                rdmas[c].append(rdma)

        for c in range(C):
            for r in rdmas[c]:
                r.wait_recv()
            g_sum = jnp.sum(comm_ref[c, :, :], axis=0)
            rows = pl.ds(c * mc, mc)
            out_ref[rows, :] = out_ref[rows, :] * (1.0 / g_sum)[:, None]

        for c in range(C):
            for r in rdmas[c]:
                r.wait_send()

        @functools.partial(
            pl.run_scoped, second_barrier=pltpu.SemaphoreType.REGULAR
        )
        def _(second_barrier):
            for peer in peers:
                pl.semaphore_signal(
                    second_barrier, inc=1,
                    device_id=(peer,), device_id_type=pl.DeviceIdType.MESH,
                )
            pl.semaphore_wait(second_barrier, N_DEV - 1)

    return pl.pallas_call(
        body,
        out_shape=jax.ShapeDtypeStruct((m, n_per), jnp.float32),
        in_specs=[pl.BlockSpec(memory_space=pltpu.MemorySpace.HBM)],
        out_specs=pl.BlockSpec(memory_space=pltpu.VMEM),
        scratch_shapes=[
            pltpu.VMEM((2, mc, n_per), jnp.float32),
            pltpu.VMEM((C, N_DEV, mc), jnp.float32),
            pltpu.SemaphoreType.DMA((2,)),
            pltpu.SemaphoreType.DMA((C, N_DEV - 1)),
            pltpu.SemaphoreType.DMA((C, N_DEV - 1)),
        ],
        compiler_params=pltpu.CompilerParams(collective_id=0),
    )(pltpu.with_memory_space_constraint(x, pltpu.MemorySpace.HBM))
